# Initial kernel scaffold; baseline (speedup 1.0000x reference)
import jax
import jax.numpy as jnp
from jax.experimental import pallas as pl


def kernel(pred, gt):
    raise NotImplementedError("write your pallas kernel here")


# speedup vs baseline: 13.7626x; 13.7626x over previous
import jax
import jax.numpy as jnp
from jax.experimental import pallas as pl
from jax.experimental.pallas import tpu as pltpu

_TAU = 0.01
_SINKHORN_ITERS = 5
_EPS = 1e-05
_TOP_K = 5

_B, _N, _M = 8, 2048, 2048
_CHUNK = 256
_NCH = _N // _CHUNK


def _apml_kernel(pred_ref, gtt_ref, out_ref, d_ref, s_ref, r_ref, c_ref):
    b = pl.program_id(0)

    gtt = gtt_ref[0]
    b2 = jnp.sum(gtt * gtt, axis=0, keepdims=True)
    bx = gtt[0:1, :]
    by = gtt[1:2, :]
    bz = gtt[2:3, :]

    def build(i, _):
        sl = pl.ds(i * _CHUNK, _CHUNK)
        a = pred_ref[0, sl, :]
        a2 = jnp.sum(a * a, axis=1, keepdims=True)
        ax = a[:, 0:1]
        ay = a[:, 1:2]
        az = a[:, 2:3]
        d2 = a2 + b2 - 2.0 * (ax * bx + ay * by + az * bz)
        dchunk = jnp.sqrt(jnp.maximum(d2, 1e-12))
        d_ref[sl, :] = dchunk
        s_ref[sl, :] = jnp.exp(dchunk * (-1.0 / _TAU))
        return 0

    jax.lax.fori_loop(0, _NCH, build, 0, unroll=False)
    r_ref[...] = jnp.ones((_N, 1), jnp.float32)
    c_ref[...] = jnp.ones((1, _M), jnp.float32)

    def sink(_, carry):
        c = c_ref[...]

        def rowup(i, _):
            sl = pl.ds(i * _CHUNK, _CHUNK)
            rs = jnp.sum(s_ref[sl, :] * c, axis=1, keepdims=True)
            r_old = r_ref[sl, :]
            r_ref[sl, :] = r_old / (r_old * rs + _EPS)
            return 0

        jax.lax.fori_loop(0, _NCH, rowup, 0, unroll=False)

        def colacc(i, acc):
            sl = pl.ds(i * _CHUNK, _CHUNK)
            return acc + jnp.sum(s_ref[sl, :] * r_ref[sl, :], axis=0,
                                 keepdims=True)

        cs = jax.lax.fori_loop(0, _NCH, colacc,
                               jnp.zeros((1, _M), jnp.float32), unroll=False)
        c_old = c_ref[...]
        c_ref[...] = c_old / (c_old * cs + _EPS)
        return carry

    jax.lax.fori_loop(0, _SINKHORN_ITERS, sink, 0, unroll=False)

    c = c_ref[...]
    iota = jax.lax.broadcasted_iota(jnp.int32, (_CHUNK, _M), 1)

    def select(i, acc_loss):
        sl = pl.ds(i * _CHUNK, _CHUNK)
        s_ref[sl, :] = s_ref[sl, :] * c
        dchunk = d_ref[sl, :]

        def topk_it(_, carry):
            s1, s2 = carry
            w = s_ref[sl, :]
            m = jnp.max(w, axis=1, keepdims=True)
            idx = jnp.min(jnp.where(w >= m, iota, _M), axis=1, keepdims=True)
            sel = iota == idx
            d_at = jnp.sum(jnp.where(sel, dchunk, 0.0), axis=1, keepdims=True)
            s_ref[sl, :] = jnp.where(sel, -jnp.inf, w)
            return s1 + m, s2 + m * d_at

        z = jnp.zeros((_CHUNK, 1), jnp.float32)
        s1, s2 = jax.lax.fori_loop(0, _TOP_K, topk_it, (z, z), unroll=False)
        rch = r_ref[sl, :]
        row_loss = (rch * s2) / (rch * s1 + _EPS)
        return acc_loss + jnp.sum(row_loss)

    loss_b = jax.lax.fori_loop(0, _NCH, select, jnp.float32(0.0), unroll=False)

    @pl.when(b == 0)
    def _():
        out_ref[...] = jnp.zeros((1, 1), jnp.float32)

    out_ref[...] = out_ref[...] + loss_b * (1.0 / _B)


def _apml(pred, gt, interpret=False):
    predp = jnp.pad(pred, ((0, 0), (0, 0), (0, 5)))
    gttp = jnp.pad(jnp.swapaxes(gt, 1, 2), ((0, 0), (0, 5), (0, 0)))
    out = pl.pallas_call(
        _apml_kernel,
        grid=(_B,),
        in_specs=[
            pl.BlockSpec((1, _N, 8), lambda b: (b, 0, 0)),
            pl.BlockSpec((1, 8, _M), lambda b: (b, 0, 0)),
        ],
        out_specs=pl.BlockSpec((1, 1), lambda b: (0, 0)),
        out_shape=jax.ShapeDtypeStruct((1, 1), jnp.float32),
        scratch_shapes=[
            pltpu.VMEM((_N, _M), jnp.float32),
            pltpu.VMEM((_N, _M), jnp.float32),
            pltpu.VMEM((_N, 1), jnp.float32),
            pltpu.VMEM((1, _M), jnp.float32),
        ],
        compiler_params=pltpu.CompilerParams(
            dimension_semantics=("arbitrary",),
        ),
        interpret=interpret,
    )(predp, gttp)
    return out[0, 0]


def kernel(pred, gt):
    return _apml(pred, gt)


# speedup vs baseline: 15.3914x; 1.1183x over previous
import jax
import jax.numpy as jnp
from jax.experimental import pallas as pl
from jax.experimental.pallas import tpu as pltpu

_TAU = 0.01
_SINKHORN_ITERS = 5
_EPS = 1e-05
_TOP_K = 5

_B, _N, _M = 8, 2048, 2048
_CHUNK = 256
_NCH = _N // _CHUNK


def _apml_kernel(pred_ref, gtt_ref, out_ref, s_ref, r_ref):
    b = pl.program_id(0)

    gtt = gtt_ref[0]
    b2 = jnp.sum(gtt * gtt, axis=0, keepdims=True)
    bx = gtt[0:1, :]
    by = gtt[1:2, :]
    bz = gtt[2:3, :]

    def build(i, colacc):
        sl = pl.ds(i * _CHUNK, _CHUNK)
        a = pred_ref[0, sl, :]
        a2 = jnp.sum(a * a, axis=1, keepdims=True)
        ax = a[:, 0:1]
        ay = a[:, 1:2]
        az = a[:, 2:3]
        d2 = a2 + b2 - 2.0 * (ax * bx + ay * by + az * bz)
        dchunk = jnp.sqrt(jnp.maximum(d2, 1e-12))
        s = jnp.exp(dchunk * (-1.0 / _TAU))
        s_ref[sl, :] = s
        rs = jnp.sum(s, axis=1, keepdims=True)
        r_new = 1.0 / (rs + _EPS)
        r_ref[sl, :] = r_new
        return colacc + jnp.sum(s * r_new, axis=0, keepdims=True)

    cs = jax.lax.fori_loop(0, _NCH, build, jnp.zeros((1, _M), jnp.float32),
                           unroll=False)
    c0 = 1.0 / (cs + _EPS)

    def sink(_, c):
        def sweep(i, colacc):
            sl = pl.ds(i * _CHUNK, _CHUNK)
            s = s_ref[sl, :]
            rs = jnp.sum(s * c, axis=1, keepdims=True)
            r_old = r_ref[sl, :]
            r_new = r_old / (r_old * rs + _EPS)
            r_ref[sl, :] = r_new
            return colacc + jnp.sum(s * r_new, axis=0, keepdims=True)

        cs = jax.lax.fori_loop(0, _NCH, sweep,
                               jnp.zeros((1, _M), jnp.float32), unroll=False)
        return c / (c * cs + _EPS)

    c = jax.lax.fori_loop(0, _SINKHORN_ITERS - 1, sink, c0, unroll=False)

    lnc = jnp.log(c)

    def select(i, acc_loss):
        sl = pl.ds(i * _CHUNK, _CHUNK)

        def tk(_, carry):
            thresh, k, s1, s2 = carry
            w = s_ref[sl, :] * c
            masked = jnp.where(w < thresh, w, -jnp.inf)
            m = jnp.max(masked, axis=1, keepdims=True)
            eq = w == m
            cnt = jnp.sum(jnp.where(eq, 1.0, 0.0), axis=1, keepdims=True)
            slnc = jnp.sum(jnp.where(eq, lnc, 0.0), axis=1, keepdims=True)
            live = (k < float(_TOP_K)) & (m > 0.0)
            sum_d = (cnt * jnp.log(m) - slnc) * (-_TAU)
            s1 = s1 + jnp.where(live, m * cnt, 0.0)
            s2 = s2 + jnp.where(live, m * sum_d, 0.0)
            k = k + jnp.where(k < float(_TOP_K), cnt, 0.0)
            return m, k, s1, s2

        z = jnp.zeros((_CHUNK, 1), jnp.float32)
        init = (jnp.full((_CHUNK, 1), jnp.inf, jnp.float32), z, z, z)
        _, _, s1, s2 = jax.lax.fori_loop(0, _TOP_K, tk, init, unroll=False)
        rch = r_ref[sl, :]
        row_loss = (rch * s2) / (rch * s1 + _EPS)
        return acc_loss + jnp.sum(row_loss)

    loss_b = jax.lax.fori_loop(0, _NCH, select, jnp.float32(0.0), unroll=False)

    @pl.when(b == 0)
    def _():
        out_ref[...] = jnp.zeros((1, 1), jnp.float32)

    out_ref[...] = out_ref[...] + loss_b * (1.0 / _B)


def _apml(pred, gt, interpret=False):
    predp = jnp.pad(pred, ((0, 0), (0, 0), (0, 5)))
    gttp = jnp.pad(jnp.swapaxes(gt, 1, 2), ((0, 0), (0, 5), (0, 0)))
    out = pl.pallas_call(
        _apml_kernel,
        grid=(_B,),
        in_specs=[
            pl.BlockSpec((1, _N, 8), lambda b: (b, 0, 0)),
            pl.BlockSpec((1, 8, _M), lambda b: (b, 0, 0)),
        ],
        out_specs=pl.BlockSpec((1, 1), lambda b: (0, 0)),
        out_shape=jax.ShapeDtypeStruct((1, 1), jnp.float32),
        scratch_shapes=[
            pltpu.VMEM((_N, _M), jnp.float32),
            pltpu.VMEM((_N, 1), jnp.float32),
        ],
        compiler_params=pltpu.CompilerParams(
            dimension_semantics=("arbitrary",),
        ),
        interpret=interpret,
    )(predp, gttp)
    return out[0, 0]


def kernel(pred, gt):
    return _apml(pred, gt)


# speedup vs baseline: 17.0083x; 1.1050x over previous
import jax
import jax.numpy as jnp
from jax.experimental import pallas as pl
from jax.experimental.pallas import tpu as pltpu

_TAU = 0.01
_SINKHORN_ITERS = 5
_EPS = 1e-05
_TOP_K = 5

_B, _N, _M = 8, 2048, 2048
_CHUNK = 256
_NCH = _N // _CHUNK


def _apml_kernel(pred_ref, gtt_ref, out_ref, s_ref, r_ref):
    b = pl.program_id(0)

    gtt = gtt_ref[0]
    b2 = jnp.sum(gtt * gtt, axis=0, keepdims=True)

    def build(i, colacc):
        sl = pl.ds(i * _CHUNK, _CHUNK)
        a = pred_ref[0, sl, :]
        a2 = jnp.sum(a * a, axis=1, keepdims=True)
        ab = jnp.dot(a, gtt, preferred_element_type=jnp.float32)
        d2 = a2 + b2 - 2.0 * ab
        dchunk = jnp.sqrt(jnp.maximum(d2, 1e-12))
        s = jnp.exp(dchunk * (-1.0 / _TAU))
        s_ref[sl, :] = s
        rs = jnp.sum(s, axis=1, keepdims=True)
        r_new = 1.0 / (rs + _EPS)
        r_ref[sl, :] = r_new
        return colacc + jnp.sum(s * r_new, axis=0, keepdims=True)

    cs = jax.lax.fori_loop(0, _NCH, build, jnp.zeros((1, _M), jnp.float32),
                           unroll=False)
    c0 = 1.0 / (cs + _EPS)

    def sink(_, c):
        def sweep(i, colacc):
            sl = pl.ds(i * _CHUNK, _CHUNK)
            s = s_ref[sl, :]
            rs = jnp.sum(s * c, axis=1, keepdims=True)
            r_old = r_ref[sl, :]
            r_new = r_old / (r_old * rs + _EPS)
            r_ref[sl, :] = r_new
            return colacc + jnp.sum(s * r_new, axis=0, keepdims=True)

        cs = jax.lax.fori_loop(0, _NCH, sweep,
                               jnp.zeros((1, _M), jnp.float32), unroll=False)
        return c / (c * cs + _EPS)

    c = jax.lax.fori_loop(0, _SINKHORN_ITERS - 1, sink, c0, unroll=False)

    lnc = jnp.log(c)

    def select(i, acc_loss):
        sl = pl.ds(i * _CHUNK, _CHUNK)
        s_ref[sl, :] = s_ref[sl, :] * c

        def tk(_, carry):
            thresh, k, s1, s2 = carry
            w = s_ref[sl, :]
            masked = jnp.where(w < thresh, w, -jnp.inf)
            m = jnp.max(masked, axis=1, keepdims=True)
            eq = w == m
            cnt = jnp.sum(jnp.where(eq, 1.0, 0.0), axis=1, keepdims=True)
            slnc = jnp.sum(jnp.where(eq, lnc, 0.0), axis=1, keepdims=True)
            live = (k < float(_TOP_K)) & (m > 0.0)
            sum_d = (cnt * jnp.log(m) - slnc) * (-_TAU)
            s1 = s1 + jnp.where(live, m * cnt, 0.0)
            s2 = s2 + jnp.where(live, m * sum_d, 0.0)
            k = k + jnp.where(k < float(_TOP_K), cnt, 0.0)
            return m, k, s1, s2

        z = jnp.zeros((_CHUNK, 1), jnp.float32)
        init = (jnp.full((_CHUNK, 1), jnp.inf, jnp.float32), z, z, z)
        _, _, s1, s2 = jax.lax.fori_loop(0, _TOP_K, tk, init, unroll=False)
        rch = r_ref[sl, :]
        row_loss = (rch * s2) / (rch * s1 + _EPS)
        return acc_loss + jnp.sum(row_loss)

    loss_b = jax.lax.fori_loop(0, _NCH, select, jnp.float32(0.0), unroll=False)

    @pl.when(b == 0)
    def _():
        out_ref[...] = jnp.zeros((1, 1), jnp.float32)

    out_ref[...] = out_ref[...] + loss_b * (1.0 / _B)


def _apml(pred, gt, interpret=False):
    predp = jnp.pad(pred, ((0, 0), (0, 0), (0, 5)))
    gttp = jnp.pad(jnp.swapaxes(gt, 1, 2), ((0, 0), (0, 5), (0, 0)))
    out = pl.pallas_call(
        _apml_kernel,
        grid=(_B,),
        in_specs=[
            pl.BlockSpec((1, _N, 8), lambda b: (b, 0, 0)),
            pl.BlockSpec((1, 8, _M), lambda b: (b, 0, 0)),
        ],
        out_specs=pl.BlockSpec((1, 1), lambda b: (0, 0)),
        out_shape=jax.ShapeDtypeStruct((1, 1), jnp.float32),
        scratch_shapes=[
            pltpu.VMEM((_N, _M), jnp.float32),
            pltpu.VMEM((_N, 1), jnp.float32),
        ],
        compiler_params=pltpu.CompilerParams(
            dimension_semantics=("arbitrary",),
        ),
        interpret=interpret,
    )(predp, gttp)
    return out[0, 0]


def kernel(pred, gt):
    return _apml(pred, gt)
